# baseline (speedup 1.0000x reference)
import jax
import jax.numpy as jnp
from jax.experimental import pallas as pl
from jax.experimental.pallas import tpu as pltpu

_B = 16
_L = 4096
_D = 1024
_CH = 512
_NQ = 4
_DEPTH = 4
_NBUF = _NQ * _DEPTH
_T_MAX = _B * (_L // _CH)


def _body(lens_ref, rows_ref, offs_ref, t_ref, x_hbm, o_ref, buf, sems):
    T = t_ref[0]
    G = jax.lax.div(T + (_NQ - 1), _NQ)

    def copy(t, k):
        slot = jax.lax.rem(t, _NBUF)
        r = rows_ref[t]
        off = pl.multiple_of(offs_ref[t], _CH)
        return pltpu.make_async_copy(
            x_hbm.at[r, pl.ds(off, _CH), :],
            buf.at[slot],
            sems.at[slot],
        )

    for g in range(_DEPTH):
        for k in range(_NQ):
            t = g * _NQ + k

            @pl.when(t < T)
            def _(t=t, k=k):
                copy(jnp.int32(t), k).start(priority=k % 2)

    o_ref[...] = jnp.zeros_like(o_ref)

    def step(g, carry):
        for k in range(_NQ):
            t = g * _NQ + k
            live = t < T

            @pl.when(live)
            def _(t=t, k=k):
                copy(t, k).wait()
                nxt = t + _NBUF

                @pl.when(nxt < T)
                def _():
                    copy(nxt, k).start(priority=k % 2)

        return carry

    jax.lax.fori_loop(0, G, step, 0)

    for i in range(_B):
        o_ref[pl.ds(i, 1), :] = (
            o_ref[pl.ds(i, 1), :] / lens_ref[i].astype(jnp.float32)
        )


@jax.jit
def kernel(V, atoms_lens):
    E = V[0]
    lens = atoms_lens.astype(jnp.int32)
    nb = (lens + _CH - 1) // _CH
    prefix = jnp.cumsum(nb).astype(jnp.int32)
    T = prefix[-1]
    t_arr = jnp.arange(_T_MAX, dtype=jnp.int32)
    row = jnp.minimum(
        jnp.searchsorted(prefix, t_arr, side="right").astype(jnp.int32), _B - 1
    )
    start = jnp.concatenate([jnp.zeros((1,), jnp.int32), prefix[:-1]])
    off = jnp.clip((t_arr - start[row]) * _CH, 0, _L - _CH)

    grid_spec = pltpu.PrefetchScalarGridSpec(
        num_scalar_prefetch=4,
        grid=(1,),
        in_specs=[pl.BlockSpec(memory_space=pl.ANY)],
        out_specs=pl.BlockSpec((_B, _D), lambda i, *_: (0, 0)),
        scratch_shapes=[
            pltpu.VMEM((_NBUF, _CH, _D), jnp.float32),
            pltpu.SemaphoreType.DMA((_NBUF,)),
        ],
    )
    return pl.pallas_call(
        _body,
        grid_spec=grid_spec,
        out_shape=jax.ShapeDtypeStruct((_B, _D), jnp.float32),
    )(lens, row, off, T.reshape(1), E)


# speedup vs baseline: 1.9444x; 1.9444x over previous
import jax
import jax.numpy as jnp
from jax.experimental import pallas as pl
from jax.experimental.pallas import tpu as pltpu

_B = 16
_L = 4096
_D = 1024
_NQ = 4
_BL = _L // _NQ


def _body(lens_ref, ieff_ref, *refs):
    xs = refs[:_NQ]
    o_ref = refs[_NQ]
    i = pl.program_id(0)
    ln = lens_ref[i]

    o_ref[...] = jnp.zeros_like(o_ref)
    for q in range(_NQ):
        rel = ln - q * _BL

        @pl.when(rel > 0)
        def _(q=q, rel=rel):
            rows = jax.lax.broadcasted_iota(jnp.int32, (1, 1, _BL, 1), 2)
            x = jnp.where(rows < rel, xs[q][...], 0.0)
            o_ref[...] += jnp.sum(x, axis=2).reshape(o_ref.shape)

    o_ref[...] = o_ref[...] / ln.astype(jnp.float32)


def _mk_map(q):
    def _map(i, lens, ieff):
        return (ieff[q, i], q, 0, 0)

    return _map


@jax.jit
def kernel(V, atoms_lens):
    E = V[0].reshape(_B, _NQ, _BL, _D)
    lens = atoms_lens.astype(jnp.int32)
    idx = jnp.arange(_B, dtype=jnp.int32)
    live = lens[None, :] > (jnp.arange(_NQ, dtype=jnp.int32) * _BL)[:, None]
    liveidx = jnp.where(live, idx[None, :], -1)
    run = jax.lax.cummax(liveidx, axis=1)
    ieff = jnp.maximum(run, 0).astype(jnp.int32)

    grid_spec = pltpu.PrefetchScalarGridSpec(
        num_scalar_prefetch=2,
        grid=(_B,),
        in_specs=[
            pl.BlockSpec((1, 1, _BL, _D), _mk_map(q)) for q in range(_NQ)
        ],
        out_specs=pl.BlockSpec((1, 1, _D), lambda i, lens, ieff: (i, 0, 0)),
    )
    out = pl.pallas_call(
        _body,
        grid_spec=grid_spec,
        out_shape=jax.ShapeDtypeStruct((_B, 1, _D), jnp.float32),
    )(lens, ieff, *([E] * _NQ))
    return out.reshape(_B, _D)


# speedup vs baseline: 2.0541x; 1.0564x over previous
import jax
import jax.numpy as jnp
from jax.experimental import pallas as pl
from jax.experimental.pallas import tpu as pltpu

_B = 16
_L = 4096
_D = 1024
_NQ = 8
_BL = _L // _NQ


def _body(lens_ref, ieff_ref, *refs):
    xs = refs[:_NQ]
    o_ref = refs[_NQ]
    i = pl.program_id(0)
    ln = lens_ref[i]

    o_ref[...] = jnp.zeros_like(o_ref)
    for q in range(_NQ):
        rel = ln - q * _BL

        @pl.when(rel > 0)
        def _(q=q, rel=rel):
            rows = jax.lax.broadcasted_iota(jnp.int32, (1, 1, _BL, 1), 2)
            x = jnp.where(rows < rel, xs[q][...], 0.0)
            o_ref[...] += jnp.sum(x, axis=2).reshape(o_ref.shape)

    o_ref[...] = o_ref[...] / ln.astype(jnp.float32)


def _mk_map(q):
    def _map(i, lens, ieff):
        return (ieff[q, i], q, 0, 0)

    return _map


@jax.jit
def kernel(V, atoms_lens):
    E = V[0].reshape(_B, _NQ, _BL, _D)
    lens = atoms_lens.astype(jnp.int32)
    idx = jnp.arange(_B, dtype=jnp.int32)
    live = lens[None, :] > (jnp.arange(_NQ, dtype=jnp.int32) * _BL)[:, None]
    liveidx = jnp.where(live, idx[None, :], -1)
    run = jax.lax.cummax(liveidx, axis=1)
    ieff = jnp.maximum(run, 0).astype(jnp.int32)

    grid_spec = pltpu.PrefetchScalarGridSpec(
        num_scalar_prefetch=2,
        grid=(_B,),
        in_specs=[
            pl.BlockSpec((1, 1, _BL, _D), _mk_map(q)) for q in range(_NQ)
        ],
        out_specs=pl.BlockSpec((1, 1, _D), lambda i, lens, ieff: (i, 0, 0)),
    )
    out = pl.pallas_call(
        _body,
        grid_spec=grid_spec,
        out_shape=jax.ShapeDtypeStruct((_B, 1, _D), jnp.float32),
    )(lens, ieff, *([E] * _NQ))
    return out.reshape(_B, _D)


# speedup vs baseline: 2.0623x; 1.0040x over previous
import jax
import jax.numpy as jnp
from jax.experimental import pallas as pl
from jax.experimental.pallas import tpu as pltpu

_B = 16
_L = 4096
_D = 1024
_NQ = 16
_BL = _L // _NQ


def _body(lens_ref, ieff_ref, *refs):
    xs = refs[:_NQ]
    o_ref = refs[_NQ]
    i = pl.program_id(0)
    ln = lens_ref[i]

    o_ref[...] = jnp.zeros_like(o_ref)
    for q in range(_NQ):
        rel = ln - q * _BL

        @pl.when(rel > 0)
        def _(q=q, rel=rel):
            rows = jax.lax.broadcasted_iota(jnp.int32, (1, 1, _BL, 1), 2)
            x = jnp.where(rows < rel, xs[q][...], 0.0)
            o_ref[...] += jnp.sum(x, axis=2).reshape(o_ref.shape)

    o_ref[...] = o_ref[...] / ln.astype(jnp.float32)


def _mk_map(q):
    def _map(i, lens, ieff):
        return (ieff[q, i], q, 0, 0)

    return _map


@jax.jit
def kernel(V, atoms_lens):
    E = V[0].reshape(_B, _NQ, _BL, _D)
    lens = atoms_lens.astype(jnp.int32)
    idx = jnp.arange(_B, dtype=jnp.int32)
    live = lens[None, :] > (jnp.arange(_NQ, dtype=jnp.int32) * _BL)[:, None]
    liveidx = jnp.where(live, idx[None, :], -1)
    run = jax.lax.cummax(liveidx, axis=1)
    ieff = jnp.maximum(run, 0).astype(jnp.int32)

    grid_spec = pltpu.PrefetchScalarGridSpec(
        num_scalar_prefetch=2,
        grid=(_B,),
        in_specs=[
            pl.BlockSpec((1, 1, _BL, _D), _mk_map(q)) for q in range(_NQ)
        ],
        out_specs=pl.BlockSpec((1, 1, _D), lambda i, lens, ieff: (i, 0, 0)),
    )
    out = pl.pallas_call(
        _body,
        grid_spec=grid_spec,
        out_shape=jax.ShapeDtypeStruct((_B, 1, _D), jnp.float32),
    )(lens, ieff, *([E] * _NQ))
    return out.reshape(_B, _D)
